# baseline (speedup 1.0000x reference)
import functools

import jax
import jax.numpy as jnp
from jax import lax
from jax.experimental import pallas as pl
from jax.experimental.pallas import tpu as pltpu
from jax.experimental.pallas import tpu_sc as plsc

N_NODES = 10000
N_EDGES = 320000
D_NODE = 128
D_EDGE = 16

NC = 2
NS = 16
NW = NC * NS
EDGES_PER_W = N_EDGES // NW
SUB = 80
SUBS = 5
CHUNK = SUB * SUBS
CHUNKS_PER_W = EDGES_PER_W // CHUNK
GROUPS = CHUNK // 16


def _lrelu(v):
    return jnp.maximum(v, v * 0.1)


def _node_proj_body(x_ref, ws_ref, wd_ref, zs_ref, zd_ref):
    xb = x_ref[...]
    zs_ref[...] = lax.dot_general(
        xb, ws_ref[...], (((1,), (0,)), ((), ())),
        preferred_element_type=jnp.float32, precision=lax.Precision.HIGHEST)
    zd_ref[...] = lax.dot_general(
        xb, wd_ref[...], (((1,), (0,)), ((), ())),
        preferred_element_type=jnp.float32, precision=lax.Precision.HIGHEST)


def _node_proj(x, ws_pad, wd_pad):
    blk = 2000
    grid = N_NODES // blk
    return pl.pallas_call(
        _node_proj_body,
        grid=(grid,),
        in_specs=[
            pl.BlockSpec((blk, D_NODE), lambda i: (i, 0)),
            pl.BlockSpec((D_NODE, 16), lambda i: (0, 0)),
            pl.BlockSpec((D_NODE, 16), lambda i: (0, 0)),
        ],
        out_specs=[
            pl.BlockSpec((blk, 16), lambda i: (i, 0)),
            pl.BlockSpec((blk, 16), lambda i: (i, 0)),
        ],
        out_shape=[
            jax.ShapeDtypeStruct((N_NODES, 16), jnp.float32),
            jax.ShapeDtypeStruct((N_NODES, 16), jnp.float32),
        ],
    )(x, ws_pad, wd_pad)


def _edge_proj_body(et_ref, wt_ref, bt_ref, z_ref):
    z = lax.dot_general(
        wt_ref[...], et_ref[...], (((1,), (0,)), ((), ())),
        preferred_element_type=jnp.float32, precision=lax.Precision.HIGHEST)
    z_ref[...] = z + bt_ref[...]


def _edge_proj_t(et, wet_pad, b7t_pad):
    blk = 12800
    grid = N_EDGES // blk
    return pl.pallas_call(
        _edge_proj_body,
        grid=(grid,),
        in_specs=[
            pl.BlockSpec((D_EDGE, blk), lambda i: (0, i)),
            pl.BlockSpec((8, D_EDGE), lambda i: (0, 0)),
            pl.BlockSpec((8, 1), lambda i: (0, 0)),
        ],
        out_specs=pl.BlockSpec((8, blk), lambda i: (0, i)),
        out_shape=jax.ShapeDtypeStruct((8, N_EDGES), jnp.float32),
    )(et, wet_pad, b7t_pad)


def _tail_body(h_ref, w8t_ref, b8_ref, w81t_ref, b81_ref, w9t_ref, b9_ref,
               out_ref):
    hp = lax.Precision.HIGHEST
    h1 = lax.dot_general(
        w8t_ref[...], h_ref[...], (((1,), (0,)), ((), ())),
        preferred_element_type=jnp.float32, precision=hp) + b8_ref[...]
    h1 = _lrelu(h1)
    h2 = lax.dot_general(
        w81t_ref[...], h1, (((1,), (0,)), ((), ())),
        preferred_element_type=jnp.float32, precision=hp) + b81_ref[...]
    h2 = _lrelu(h2)
    o = lax.dot_general(
        w9t_ref[...], h2, (((1,), (0,)), ((), ())),
        preferred_element_type=jnp.float32, precision=hp) + b9_ref[...]
    o0 = o[0:1, :]
    o1 = o[1:2, :]
    m = jnp.maximum(o0, o1)
    e0 = jnp.exp(o0 - m)
    e1 = jnp.exp(o1 - m)
    inv = 1.0 / (e0 + e1)
    out_ref[...] = jnp.concatenate([e0 * inv, e1 * inv], axis=0)


def _tail(h0t, w8t, b8c, w81t, b81c, w9t, b9c):
    blk = 12800
    grid = N_EDGES // blk
    return pl.pallas_call(
        _tail_body,
        grid=(grid,),
        in_specs=[
            pl.BlockSpec((6, blk), lambda i: (0, i)),
            pl.BlockSpec((12, 6), lambda i: (0, 0)),
            pl.BlockSpec((12, 1), lambda i: (0, 0)),
            pl.BlockSpec((6, 12), lambda i: (0, 0)),
            pl.BlockSpec((6, 1), lambda i: (0, 0)),
            pl.BlockSpec((2, 6), lambda i: (0, 0)),
            pl.BlockSpec((2, 1), lambda i: (0, 0)),
        ],
        out_specs=pl.BlockSpec((2, blk), lambda i: (0, i)),
        out_shape=jax.ShapeDtypeStruct((2, N_EDGES), jnp.float32),
    )(h0t, w8t, b8c, w81t, b81c, w9t, b9c)


def _sc_body(zs_hbm, zd_hbm, zet_hbm, src_hbm, dst_hbm, h0t_hbm,
             idx_s, idx_d, buf_a, buf_b, ze_buf, out_buf,
             sem_g, sem_o):
    wid = lax.axis_index("s") * NC + lax.axis_index("c")

    iota16 = lax.iota(jnp.int32, 16)
    col_idx = [jnp.full((16,), k, jnp.int32) for k in range(6)]

    def start_fetch(j, par):
        gid = wid * CHUNKS_PER_W + j
        r0 = gid * CHUNK
        pltpu.sync_copy(src_hbm.at[pl.ds(gid * SUBS, SUBS)], idx_s.at[par])
        pltpu.sync_copy(dst_hbm.at[pl.ds(gid * SUBS, SUBS)], idx_d.at[par])
        for k in range(SUBS):
            pltpu.async_copy(
                zs_hbm.at[idx_s.at[par].at[k]],
                buf_a.at[par].at[pl.ds(k * SUB, SUB)], sem_g.at[par])
            pltpu.async_copy(
                zd_hbm.at[idx_d.at[par].at[k]],
                buf_b.at[par].at[pl.ds(k * SUB, SUB)], sem_g.at[par])
        pltpu.async_copy(
            zet_hbm.at[:, pl.ds(r0, CHUNK)], ze_buf.at[par], sem_g.at[par])

    def drain_fetch(par):
        for k in range(SUBS):
            pltpu.make_async_copy(
                zs_hbm.at[idx_s.at[par].at[k]],
                buf_a.at[par].at[pl.ds(k * SUB, SUB)], sem_g.at[par]).wait()
            pltpu.make_async_copy(
                zd_hbm.at[idx_d.at[par].at[k]],
                buf_b.at[par].at[pl.ds(k * SUB, SUB)], sem_g.at[par]).wait()
        pltpu.make_async_copy(
            zet_hbm.at[:, pl.ds(0, CHUNK)], ze_buf.at[par],
            sem_g.at[par]).wait()

    def drain_out(j, par):
        r0 = (wid * CHUNKS_PER_W + j) * CHUNK
        pltpu.make_async_copy(
            out_buf.at[par], h0t_hbm.at[:, pl.ds(r0, CHUNK)],
            sem_o.at[par]).wait()

    start_fetch(0, 0)

    def chunk_body(j, carry):
        par = lax.rem(j, 2)
        nxt = lax.rem(j + 1, 2)
        gid = wid * CHUNKS_PER_W + j
        r0 = gid * CHUNK

        @pl.when(j + 1 < CHUNKS_PER_W)
        def _():
            start_fetch(j + 1, nxt)

        drain_fetch(par)

        bfa = buf_a.at[par]
        bfb = buf_b.at[par]
        zeb = ze_buf.at[par]
        obf = out_buf.at[par]

        def group_body(g, gcarry):
            rows = g * 16 + iota16
            g16 = g * 16
            for k in range(6):
                a = plsc.load_gather(bfa, [rows, col_idx[k]])
                b = plsc.load_gather(bfb, [rows, col_idx[k]])
                z = zeb[k, pl.ds(g16, 16)]
                obf[k, pl.ds(g16, 16)] = _lrelu(a + b + z)
            return gcarry

        lax.fori_loop(0, GROUPS, group_body, 0)

        @pl.when(j >= 2)
        def _():
            drain_out(j - 2, par)

        pltpu.async_copy(obf, h0t_hbm.at[:, pl.ds(r0, CHUNK)], sem_o.at[par])
        return carry

    lax.fori_loop(0, CHUNKS_PER_W, chunk_body, 0)
    drain_out(CHUNKS_PER_W - 2, lax.rem(CHUNKS_PER_W - 2, 2))
    drain_out(CHUNKS_PER_W - 1, lax.rem(CHUNKS_PER_W - 1, 2))


def _sc_gather_h0(zs, zd, zet, src2d, dst2d):
    mesh = plsc.VectorSubcoreMesh(core_axis_name="c", subcore_axis_name="s")
    fn = functools.partial(
        pl.kernel,
        out_type=jax.ShapeDtypeStruct((6, N_EDGES), jnp.float32),
        mesh=mesh,
        compiler_params=pltpu.CompilerParams(
            needs_layout_passes=False, use_tc_tiling_on_sc=False),
        scratch_types=[
            pltpu.VMEM((2, SUBS, SUB), jnp.int32),
            pltpu.VMEM((2, SUBS, SUB), jnp.int32),
            pltpu.VMEM((2, CHUNK, 16), jnp.float32),
            pltpu.VMEM((2, CHUNK, 16), jnp.float32),
            pltpu.VMEM((2, 8, CHUNK), jnp.float32),
            pltpu.VMEM((2, 6, CHUNK), jnp.float32),
            pltpu.SemaphoreType.DMA((2,)),
            pltpu.SemaphoreType.DMA((2,)),
        ],
    )(_sc_body)
    return fn(zs, zd, zet, src2d, dst2d)


def kernel(x, e, edge_index, W7, b7, W8, b8, W81, b81, W9, b9):
    src2d = edge_index[0].astype(jnp.int32).reshape(N_EDGES // SUB, SUB)
    dst2d = edge_index[1].astype(jnp.int32).reshape(N_EDGES // SUB, SUB)

    ws_pad = jnp.zeros((D_NODE, 16), jnp.float32).at[:, :6].set(W7[:D_NODE])
    wd_pad = jnp.zeros((D_NODE, 16), jnp.float32).at[:, :6].set(
        W7[D_NODE:2 * D_NODE])
    wet_pad = jnp.zeros((8, D_EDGE), jnp.float32).at[:6, :].set(
        W7[2 * D_NODE:].T)
    b7t_pad = jnp.zeros((8, 1), jnp.float32).at[:6, 0].set(b7)

    zs, zd = _node_proj(x, ws_pad, wd_pad)
    zet = _edge_proj_t(e.T, wet_pad, b7t_pad)

    h0t = _sc_gather_h0(zs, zd, zet, src2d, dst2d)

    p = _tail(h0t, W8.T, b8[:, None], W81.T, b81[:, None], W9.T, b9[:, None])
    return p.T


# speedup vs baseline: 1.9950x; 1.9950x over previous
import functools

import jax
import jax.numpy as jnp
from jax import lax
from jax.experimental import pallas as pl
from jax.experimental.pallas import tpu as pltpu
from jax.experimental.pallas import tpu_sc as plsc

N_NODES = 10000
N_EDGES = 320000
D_NODE = 128
D_EDGE = 16

NC = 2
NS = 16
NW = NC * NS
EDGES_PER_W = N_EDGES // NW
CHUNK = 400
CHUNKS_PER_W = EDGES_PER_W // CHUNK
GROUPS = CHUNK // 16


def _lrelu(v):
    return jnp.maximum(v, v * 0.1)


def _node_proj_body(x_ref, ws_ref, wd_ref, zs_ref, zd_ref):
    xb = x_ref[...]
    zs_ref[...] = lax.dot_general(
        xb, ws_ref[...], (((1,), (0,)), ((), ())),
        preferred_element_type=jnp.float32, precision=lax.Precision.HIGHEST)
    zd_ref[...] = lax.dot_general(
        xb, wd_ref[...], (((1,), (0,)), ((), ())),
        preferred_element_type=jnp.float32, precision=lax.Precision.HIGHEST)


def _node_proj(x, ws_pad, wd_pad):
    blk = 2000
    grid = N_NODES // blk
    return pl.pallas_call(
        _node_proj_body,
        grid=(grid,),
        in_specs=[
            pl.BlockSpec((blk, D_NODE), lambda i: (i, 0)),
            pl.BlockSpec((D_NODE, 6), lambda i: (0, 0)),
            pl.BlockSpec((D_NODE, 6), lambda i: (0, 0)),
        ],
        out_specs=[
            pl.BlockSpec((blk, 6), lambda i: (i, 0)),
            pl.BlockSpec((blk, 6), lambda i: (i, 0)),
        ],
        out_shape=[
            jax.ShapeDtypeStruct((N_NODES, 6), jnp.float32),
            jax.ShapeDtypeStruct((N_NODES, 6), jnp.float32),
        ],
    )(x, ws_pad, wd_pad)


def _edge_proj_body(et_ref, wt_ref, bt_ref, z_ref):
    z = lax.dot_general(
        wt_ref[...], et_ref[...], (((1,), (0,)), ((), ())),
        preferred_element_type=jnp.float32, precision=lax.Precision.HIGHEST)
    z_ref[...] = z + bt_ref[...]


def _edge_proj_t(et, wet, b7t):
    blk = 32000
    grid = N_EDGES // blk
    return pl.pallas_call(
        _edge_proj_body,
        grid=(grid,),
        in_specs=[
            pl.BlockSpec((D_EDGE, blk), lambda i: (0, i)),
            pl.BlockSpec((6, D_EDGE), lambda i: (0, 0)),
            pl.BlockSpec((6, 1), lambda i: (0, 0)),
        ],
        out_specs=pl.BlockSpec((6, blk), lambda i: (0, i)),
        out_shape=jax.ShapeDtypeStruct((6, N_EDGES), jnp.float32),
    )(et, wet, b7t)


def _tail_body(h_ref, ze_ref, w8t_ref, b8_ref, w81t_ref, b81_ref,
               w9t_ref, b9_ref, out_ref):
    hp = lax.Precision.DEFAULT
    h0 = _lrelu(h_ref[...] + ze_ref[...])
    h1 = lax.dot_general(
        w8t_ref[...], h0, (((1,), (0,)), ((), ())),
        preferred_element_type=jnp.float32, precision=hp) + b8_ref[...]
    h1 = _lrelu(h1)
    h2 = lax.dot_general(
        w81t_ref[...], h1, (((1,), (0,)), ((), ())),
        preferred_element_type=jnp.float32, precision=hp) + b81_ref[...]
    h2 = _lrelu(h2)
    o = lax.dot_general(
        w9t_ref[...], h2, (((1,), (0,)), ((), ())),
        preferred_element_type=jnp.float32, precision=hp) + b9_ref[...]
    o0 = o[0:1, :]
    o1 = o[1:2, :]
    m = jnp.maximum(o0, o1)
    e0 = jnp.exp(o0 - m)
    e1 = jnp.exp(o1 - m)
    inv = 1.0 / (e0 + e1)
    out_ref[...] = jnp.concatenate([e0 * inv, e1 * inv], axis=0)


def _tail(h0t, zet, w8t, b8c, w81t, b81c, w9t, b9c):
    blk = 32000
    grid = N_EDGES // blk
    return pl.pallas_call(
        _tail_body,
        grid=(grid,),
        in_specs=[
            pl.BlockSpec((6, blk), lambda i: (0, i)),
            pl.BlockSpec((6, blk), lambda i: (0, i)),
            pl.BlockSpec((12, 6), lambda i: (0, 0)),
            pl.BlockSpec((12, 1), lambda i: (0, 0)),
            pl.BlockSpec((6, 12), lambda i: (0, 0)),
            pl.BlockSpec((6, 1), lambda i: (0, 0)),
            pl.BlockSpec((2, 6), lambda i: (0, 0)),
            pl.BlockSpec((2, 1), lambda i: (0, 0)),
        ],
        out_specs=pl.BlockSpec((2, blk), lambda i: (0, i)),
        out_shape=jax.ShapeDtypeStruct((2, N_EDGES), jnp.float32),
    )(h0t, zet, w8t, b8c, w81t, b81c, w9t, b9c)


def _sc_body(zs_hbm, zd_hbm, src_hbm, dst_hbm, h0t_hbm,
             tbl_s, tbl_d, idx_s, idx_d, out_buf, sem_g, sem_o):
    wid = lax.axis_index("s") * NC + lax.axis_index("c")

    pltpu.sync_copy(zs_hbm, tbl_s)
    pltpu.sync_copy(zd_hbm, tbl_d)

    def start_fetch(j, par):
        gid = wid * CHUNKS_PER_W + j
        r0 = gid * CHUNK
        pltpu.async_copy(src_hbm.at[pl.ds(r0, CHUNK)], idx_s.at[par],
                         sem_g.at[par])
        pltpu.async_copy(dst_hbm.at[pl.ds(r0, CHUNK)], idx_d.at[par],
                         sem_g.at[par])

    def drain_fetch(par):
        pltpu.make_async_copy(src_hbm.at[pl.ds(0, CHUNK)], idx_s.at[par],
                              sem_g.at[par]).wait()
        pltpu.make_async_copy(dst_hbm.at[pl.ds(0, CHUNK)], idx_d.at[par],
                              sem_g.at[par]).wait()

    def drain_out(j, par):
        r0 = (wid * CHUNKS_PER_W + j) * CHUNK
        pltpu.make_async_copy(
            out_buf.at[par], h0t_hbm.at[:, pl.ds(r0, CHUNK)],
            sem_o.at[par]).wait()

    start_fetch(0, 0)

    def chunk_body(j, carry):
        par = lax.rem(j, 2)
        nxt = lax.rem(j + 1, 2)
        gid = wid * CHUNKS_PER_W + j
        r0 = gid * CHUNK

        @pl.when(j + 1 < CHUNKS_PER_W)
        def _():
            start_fetch(j + 1, nxt)

        drain_fetch(par)

        isb = idx_s.at[par]
        idb = idx_d.at[par]
        obf = out_buf.at[par]

        def group_body(g, gcarry):
            g16 = g * 16
            iv_s = isb[pl.ds(g16, 16)] * 6
            iv_d = idb[pl.ds(g16, 16)] * 6
            for k in range(6):
                a = plsc.load_gather(tbl_s, [iv_s + k])
                b = plsc.load_gather(tbl_d, [iv_d + k])
                obf[k, pl.ds(g16, 16)] = a + b
            return gcarry

        lax.fori_loop(0, GROUPS, group_body, 0)

        @pl.when(j >= 2)
        def _():
            drain_out(j - 2, par)

        pltpu.async_copy(obf, h0t_hbm.at[:, pl.ds(r0, CHUNK)], sem_o.at[par])
        return carry

    lax.fori_loop(0, CHUNKS_PER_W, chunk_body, 0)
    drain_out(CHUNKS_PER_W - 2, lax.rem(CHUNKS_PER_W - 2, 2))
    drain_out(CHUNKS_PER_W - 1, lax.rem(CHUNKS_PER_W - 1, 2))


def _sc_gather_sum(zs, zd, src1, dst1):
    mesh = plsc.VectorSubcoreMesh(core_axis_name="c", subcore_axis_name="s")
    fn = functools.partial(
        pl.kernel,
        out_type=jax.ShapeDtypeStruct((6, N_EDGES), jnp.float32),
        mesh=mesh,
        compiler_params=pltpu.CompilerParams(
            needs_layout_passes=False, use_tc_tiling_on_sc=False),
        scratch_types=[
            pltpu.VMEM((N_NODES * 6,), jnp.float32),
            pltpu.VMEM((N_NODES * 6,), jnp.float32),
            pltpu.VMEM((2, CHUNK), jnp.int32),
            pltpu.VMEM((2, CHUNK), jnp.int32),
            pltpu.VMEM((2, 6, CHUNK), jnp.float32),
            pltpu.SemaphoreType.DMA((2,)),
            pltpu.SemaphoreType.DMA((2,)),
        ],
    )(_sc_body)
    return fn(zs, zd, src1, dst1)


def kernel(x, e, edge_index, W7, b7, W8, b8, W81, b81, W9, b9):
    src1 = edge_index[0].astype(jnp.int32)
    dst1 = edge_index[1].astype(jnp.int32)

    ws = W7[:D_NODE]
    wd = W7[D_NODE:2 * D_NODE]
    wet = W7[2 * D_NODE:].T
    b7t = b7[:, None]

    zs, zd = _node_proj(x, ws, wd)
    zet = _edge_proj_t(e.T, wet, b7t)

    h0t = _sc_gather_sum(zs.reshape(-1), zd.reshape(-1), src1, dst1)

    p = _tail(h0t, zet, W8.T, b8[:, None], W81.T, b81[:, None], W9.T,
              b9[:, None])
    return p.T


# speedup vs baseline: 2.3162x; 1.1610x over previous
import functools

import jax
import jax.numpy as jnp
from jax import lax
from jax.experimental import pallas as pl
from jax.experimental.pallas import tpu as pltpu
from jax.experimental.pallas import tpu_sc as plsc

N_NODES = 10000
N_EDGES = 320000
D_NODE = 128
D_EDGE = 16

NC = 2
NS = 16
NW = NC * NS
EDGES_PER_W = N_EDGES // NW
CHUNK = 400
CHUNKS_PER_W = EDGES_PER_W // CHUNK
GROUPS = CHUNK // 16


def _lrelu(v):
    return jnp.maximum(v, v * 0.1)


def _node_proj_body(x_ref, w_ref, z_ref):
    z_ref[...] = lax.dot_general(
        x_ref[...], w_ref[...], (((1,), (0,)), ((), ())),
        preferred_element_type=jnp.float32, precision=lax.Precision.HIGHEST)


def _node_proj(x, wsd):
    blk = 2000
    grid = N_NODES // blk
    return pl.pallas_call(
        _node_proj_body,
        grid=(grid,),
        in_specs=[
            pl.BlockSpec((blk, D_NODE), lambda i: (i, 0)),
            pl.BlockSpec((D_NODE, 12), lambda i: (0, 0)),
        ],
        out_specs=pl.BlockSpec((blk, 12), lambda i: (i, 0)),
        out_shape=jax.ShapeDtypeStruct((N_NODES, 12), jnp.float32),
    )(x, wsd)


def _edge_proj_body(et_ref, wt_ref, bt_ref, z_ref):
    z = lax.dot_general(
        wt_ref[...], et_ref[...], (((1,), (0,)), ((), ())),
        preferred_element_type=jnp.float32, precision=lax.Precision.HIGHEST)
    z_ref[...] = z + bt_ref[...]


def _edge_proj_t(et, wet, b7t):
    blk = 32000
    grid = N_EDGES // blk
    return pl.pallas_call(
        _edge_proj_body,
        grid=(grid,),
        in_specs=[
            pl.BlockSpec((D_EDGE, blk), lambda i: (0, i)),
            pl.BlockSpec((6, D_EDGE), lambda i: (0, 0)),
            pl.BlockSpec((6, 1), lambda i: (0, 0)),
        ],
        out_specs=pl.BlockSpec((6, blk), lambda i: (0, i)),
        out_shape=jax.ShapeDtypeStruct((6, N_EDGES), jnp.float32),
    )(et, wet, b7t)


def _tail_body(h_ref, ze_ref, w8t_ref, b8_ref, w81t_ref, b81_ref,
               w9t_ref, b9_ref, out_ref):
    hp = lax.Precision.DEFAULT
    h0 = _lrelu(h_ref[...] + ze_ref[...])
    h1 = lax.dot_general(
        w8t_ref[...], h0, (((1,), (0,)), ((), ())),
        preferred_element_type=jnp.float32, precision=hp) + b8_ref[...]
    h1 = _lrelu(h1)
    h2 = lax.dot_general(
        w81t_ref[...], h1, (((1,), (0,)), ((), ())),
        preferred_element_type=jnp.float32, precision=hp) + b81_ref[...]
    h2 = _lrelu(h2)
    o = lax.dot_general(
        w9t_ref[...], h2, (((1,), (0,)), ((), ())),
        preferred_element_type=jnp.float32, precision=hp) + b9_ref[...]
    o0 = o[0:1, :]
    o1 = o[1:2, :]
    m = jnp.maximum(o0, o1)
    e0 = jnp.exp(o0 - m)
    e1 = jnp.exp(o1 - m)
    inv = 1.0 / (e0 + e1)
    out_ref[...] = jnp.concatenate([e0 * inv, e1 * inv], axis=0)


def _tail(h0t, zet, w8t, b8c, w81t, b81c, w9t, b9c):
    blk = 32000
    grid = N_EDGES // blk
    return pl.pallas_call(
        _tail_body,
        grid=(grid,),
        in_specs=[
            pl.BlockSpec((6, blk), lambda i: (0, i)),
            pl.BlockSpec((6, blk), lambda i: (0, i)),
            pl.BlockSpec((12, 6), lambda i: (0, 0)),
            pl.BlockSpec((12, 1), lambda i: (0, 0)),
            pl.BlockSpec((6, 12), lambda i: (0, 0)),
            pl.BlockSpec((6, 1), lambda i: (0, 0)),
            pl.BlockSpec((2, 6), lambda i: (0, 0)),
            pl.BlockSpec((2, 1), lambda i: (0, 0)),
        ],
        out_specs=pl.BlockSpec((2, blk), lambda i: (0, i)),
        out_shape=jax.ShapeDtypeStruct((2, N_EDGES), jnp.float32),
    )(h0t, zet, w8t, b8c, w81t, b81c, w9t, b9c)


def _sc_body(zsd_hbm, ei_hbm, h0t_hbm,
             tbl, idx_s, idx_d, out_buf, sem_g, sem_o):
    wid = lax.axis_index("s") * NC + lax.axis_index("c")

    pltpu.sync_copy(zsd_hbm, tbl)

    def start_fetch(j, par):
        gid = wid * CHUNKS_PER_W + j
        r0 = gid * CHUNK
        pltpu.async_copy(ei_hbm.at[0, pl.ds(r0, CHUNK)], idx_s.at[par],
                         sem_g.at[par])
        pltpu.async_copy(ei_hbm.at[1, pl.ds(r0, CHUNK)], idx_d.at[par],
                         sem_g.at[par])

    def drain_fetch(par):
        pltpu.make_async_copy(ei_hbm.at[0, pl.ds(0, CHUNK)], idx_s.at[par],
                              sem_g.at[par]).wait()
        pltpu.make_async_copy(ei_hbm.at[1, pl.ds(0, CHUNK)], idx_d.at[par],
                              sem_g.at[par]).wait()

    def drain_out(j, par):
        r0 = (wid * CHUNKS_PER_W + j) * CHUNK
        pltpu.make_async_copy(
            out_buf.at[par], h0t_hbm.at[:, pl.ds(r0, CHUNK)],
            sem_o.at[par]).wait()

    start_fetch(0, 0)

    def chunk_body(j, carry):
        par = lax.rem(j, 2)
        nxt = lax.rem(j + 1, 2)
        gid = wid * CHUNKS_PER_W + j
        r0 = gid * CHUNK

        @pl.when(j + 1 < CHUNKS_PER_W)
        def _():
            start_fetch(j + 1, nxt)

        drain_fetch(par)

        isb = idx_s.at[par]
        idb = idx_d.at[par]
        obf = out_buf.at[par]

        def group_body(g, gcarry):
            g16 = g * 16
            iv_s = isb[pl.ds(g16, 16)] * 12
            iv_d = idb[pl.ds(g16, 16)] * 12
            for k in range(6):
                a = plsc.load_gather(tbl, [iv_s + k])
                b = plsc.load_gather(tbl, [iv_d + (6 + k)])
                obf[k, pl.ds(g16, 16)] = a + b
            return gcarry

        lax.fori_loop(0, GROUPS, group_body, 0)

        @pl.when(j >= 2)
        def _():
            drain_out(j - 2, par)

        pltpu.async_copy(obf, h0t_hbm.at[:, pl.ds(r0, CHUNK)], sem_o.at[par])
        return carry

    lax.fori_loop(0, CHUNKS_PER_W, chunk_body, 0)
    drain_out(CHUNKS_PER_W - 2, lax.rem(CHUNKS_PER_W - 2, 2))
    drain_out(CHUNKS_PER_W - 1, lax.rem(CHUNKS_PER_W - 1, 2))


def _sc_gather_sum(zsd, ei):
    mesh = plsc.VectorSubcoreMesh(core_axis_name="c", subcore_axis_name="s")
    fn = functools.partial(
        pl.kernel,
        out_type=jax.ShapeDtypeStruct((6, N_EDGES), jnp.float32),
        mesh=mesh,
        compiler_params=pltpu.CompilerParams(
            needs_layout_passes=False, use_tc_tiling_on_sc=False),
        scratch_types=[
            pltpu.VMEM((N_NODES * 12,), jnp.float32),
            pltpu.VMEM((2, CHUNK), jnp.int32),
            pltpu.VMEM((2, CHUNK), jnp.int32),
            pltpu.VMEM((2, 6, CHUNK), jnp.float32),
            pltpu.SemaphoreType.DMA((2,)),
            pltpu.SemaphoreType.DMA((2,)),
        ],
    )(_sc_body)
    return fn(zsd, ei)


def kernel(x, e, edge_index, W7, b7, W8, b8, W81, b81, W9, b9):
    ei = edge_index.astype(jnp.int32)

    wsd = W7[:2 * D_NODE].reshape(2, D_NODE, 6).transpose(1, 0, 2).reshape(
        D_NODE, 12)
    wet = W7[2 * D_NODE:].T
    b7t = b7[:, None]

    zsd = _node_proj(x, wsd)
    zet = _edge_proj_t(e.T, wet, b7t)

    h0t = _sc_gather_sum(zsd.reshape(-1), ei)

    p = _tail(h0t, zet, W8.T, b8[:, None], W81.T, b81[:, None], W9.T,
              b9[:, None])
    return p.T


# speedup vs baseline: 2.5665x; 1.1081x over previous
import functools

import jax
import jax.numpy as jnp
from jax import lax
from jax.experimental import pallas as pl
from jax.experimental.pallas import tpu as pltpu
from jax.experimental.pallas import tpu_sc as plsc

N_NODES = 10000
N_EDGES = 320000
D_NODE = 128
D_EDGE = 16

NC = 2
NS = 16
NW = NC * NS
EDGES_PER_W = N_EDGES // NW
CHUNK = 400
CHUNKS_PER_W = EDGES_PER_W // CHUNK
GROUPS = CHUNK // 16


def _lrelu(v):
    return jnp.maximum(v, v * 0.1)


def _node_proj_body(x_ref, w_ref, z_ref):
    z_ref[...] = lax.dot_general(
        w_ref[...], x_ref[...], (((0,), (1,)), ((), ())),
        preferred_element_type=jnp.float32, precision=lax.Precision.HIGHEST)


def _node_proj_t(x, wsd):
    return pl.pallas_call(
        _node_proj_body,
        grid=(1,),
        in_specs=[
            pl.BlockSpec((N_NODES, D_NODE), lambda i: (0, 0)),
            pl.BlockSpec((D_NODE, 12), lambda i: (0, 0)),
        ],
        out_specs=pl.BlockSpec((12, N_NODES), lambda i: (0, 0)),
        out_shape=jax.ShapeDtypeStruct((12, N_NODES), jnp.float32),
    )(x, wsd)


def _edge_proj_body(et_ref, wt_ref, bt_ref, z_ref):
    z = lax.dot_general(
        wt_ref[...], et_ref[...], (((1,), (0,)), ((), ())),
        preferred_element_type=jnp.float32, precision=lax.Precision.HIGHEST)
    z_ref[...] = z + bt_ref[...]


def _edge_proj_t(et, wet, b7t):
    blk = 32000
    grid = N_EDGES // blk
    return pl.pallas_call(
        _edge_proj_body,
        grid=(grid,),
        in_specs=[
            pl.BlockSpec((D_EDGE, blk), lambda i: (0, i)),
            pl.BlockSpec((6, D_EDGE), lambda i: (0, 0)),
            pl.BlockSpec((6, 1), lambda i: (0, 0)),
        ],
        out_specs=pl.BlockSpec((6, blk), lambda i: (0, i)),
        out_shape=jax.ShapeDtypeStruct((6, N_EDGES), jnp.float32),
    )(et, wet, b7t)


def _tail_body(h_ref, ze_ref, w8t_ref, b8_ref, w81t_ref, b81_ref,
               w9t_ref, b9_ref, out_ref):
    hp = lax.Precision.DEFAULT
    h0 = _lrelu(h_ref[...] + ze_ref[...])
    h1 = lax.dot_general(
        w8t_ref[...], h0, (((1,), (0,)), ((), ())),
        preferred_element_type=jnp.float32, precision=hp) + b8_ref[...]
    h1 = _lrelu(h1)
    h2 = lax.dot_general(
        w81t_ref[...], h1, (((1,), (0,)), ((), ())),
        preferred_element_type=jnp.float32, precision=hp) + b81_ref[...]
    h2 = _lrelu(h2)
    o = lax.dot_general(
        w9t_ref[...], h2, (((1,), (0,)), ((), ())),
        preferred_element_type=jnp.float32, precision=hp) + b9_ref[...]
    o0 = o[0:1, :]
    o1 = o[1:2, :]
    m = jnp.maximum(o0, o1)
    e0 = jnp.exp(o0 - m)
    e1 = jnp.exp(o1 - m)
    inv = 1.0 / (e0 + e1)
    out_ref[...] = jnp.concatenate([e0 * inv, e1 * inv], axis=0)


def _tail(h0t, zet, w8t, b8c, w81t, b81c, w9t, b9c):
    blk = 32000
    grid = N_EDGES // blk
    return pl.pallas_call(
        _tail_body,
        grid=(grid,),
        in_specs=[
            pl.BlockSpec((6, blk), lambda i: (0, i)),
            pl.BlockSpec((6, blk), lambda i: (0, i)),
            pl.BlockSpec((12, 6), lambda i: (0, 0)),
            pl.BlockSpec((12, 1), lambda i: (0, 0)),
            pl.BlockSpec((6, 12), lambda i: (0, 0)),
            pl.BlockSpec((6, 1), lambda i: (0, 0)),
            pl.BlockSpec((2, 6), lambda i: (0, 0)),
            pl.BlockSpec((2, 1), lambda i: (0, 0)),
        ],
        out_specs=pl.BlockSpec((2, blk), lambda i: (0, i)),
        out_shape=jax.ShapeDtypeStruct((2, N_EDGES), jnp.float32),
    )(h0t, zet, w8t, b8c, w81t, b81c, w9t, b9c)


def _sc_body(zsd_hbm, ei_hbm, h0t_hbm,
             tbl, idx_s, idx_d, out_buf, sem_g, sem_o):
    wid = lax.axis_index("s") * NC + lax.axis_index("c")

    pltpu.sync_copy(zsd_hbm, tbl)

    def start_fetch(j, par):
        gid = wid * CHUNKS_PER_W + j
        r0 = gid * CHUNK
        pltpu.async_copy(ei_hbm.at[0, pl.ds(r0, CHUNK)], idx_s.at[par],
                         sem_g.at[par])
        pltpu.async_copy(ei_hbm.at[1, pl.ds(r0, CHUNK)], idx_d.at[par],
                         sem_g.at[par])

    def drain_fetch(par):
        pltpu.make_async_copy(ei_hbm.at[0, pl.ds(0, CHUNK)], idx_s.at[par],
                              sem_g.at[par]).wait()
        pltpu.make_async_copy(ei_hbm.at[1, pl.ds(0, CHUNK)], idx_d.at[par],
                              sem_g.at[par]).wait()

    def drain_out(j, par):
        r0 = (wid * CHUNKS_PER_W + j) * CHUNK
        pltpu.make_async_copy(
            out_buf.at[par], h0t_hbm.at[:, pl.ds(r0, CHUNK)],
            sem_o.at[par]).wait()

    start_fetch(0, 0)

    def chunk_body(j, carry):
        par = lax.rem(j, 2)
        nxt = lax.rem(j + 1, 2)
        gid = wid * CHUNKS_PER_W + j
        r0 = gid * CHUNK

        @pl.when(j + 1 < CHUNKS_PER_W)
        def _():
            start_fetch(j + 1, nxt)

        drain_fetch(par)

        isb = idx_s.at[par]
        idb = idx_d.at[par]
        obf = out_buf.at[par]

        def group_body(g, gcarry):
            g16 = g * 16
            iv_s = isb[pl.ds(g16, 16)]
            iv_d = idb[pl.ds(g16, 16)]
            for k in range(6):
                a = plsc.load_gather(tbl, [iv_s + k * N_NODES])
                b = plsc.load_gather(tbl, [iv_d + (6 + k) * N_NODES])
                obf[k, pl.ds(g16, 16)] = a + b
            return gcarry

        lax.fori_loop(0, GROUPS, group_body, 0)

        @pl.when(j >= 2)
        def _():
            drain_out(j - 2, par)

        pltpu.async_copy(obf, h0t_hbm.at[:, pl.ds(r0, CHUNK)], sem_o.at[par])
        return carry

    lax.fori_loop(0, CHUNKS_PER_W, chunk_body, 0)
    drain_out(CHUNKS_PER_W - 2, lax.rem(CHUNKS_PER_W - 2, 2))
    drain_out(CHUNKS_PER_W - 1, lax.rem(CHUNKS_PER_W - 1, 2))


def _sc_gather_sum(zsd, ei):
    mesh = plsc.VectorSubcoreMesh(core_axis_name="c", subcore_axis_name="s")
    fn = functools.partial(
        pl.kernel,
        out_type=jax.ShapeDtypeStruct((6, N_EDGES), jnp.float32),
        mesh=mesh,
        compiler_params=pltpu.CompilerParams(
            needs_layout_passes=False, use_tc_tiling_on_sc=False),
        scratch_types=[
            pltpu.VMEM((N_NODES * 12,), jnp.float32),
            pltpu.VMEM((2, CHUNK), jnp.int32),
            pltpu.VMEM((2, CHUNK), jnp.int32),
            pltpu.VMEM((2, 6, CHUNK), jnp.float32),
            pltpu.SemaphoreType.DMA((2,)),
            pltpu.SemaphoreType.DMA((2,)),
        ],
    )(_sc_body)
    return fn(zsd, ei)


def kernel(x, e, edge_index, W7, b7, W8, b8, W81, b81, W9, b9):
    ei = edge_index.astype(jnp.int32)

    wsd = W7[:2 * D_NODE].reshape(2, D_NODE, 6).transpose(1, 0, 2).reshape(
        D_NODE, 12)
    wet = W7[2 * D_NODE:].T
    b7t = b7[:, None]

    zsdt = _node_proj_t(x, wsd)
    zet = _edge_proj_t(e.T, wet, b7t)

    h0t = _sc_gather_sum(zsdt.reshape(-1), ei)

    p = _tail(h0t, zet, W8.T, b8[:, None], W81.T, b81[:, None], W9.T,
              b9[:, None])
    return p.T
